# Initial kernel scaffold; baseline (speedup 1.0000x reference)
import jax
import jax.numpy as jnp
from jax.experimental import pallas as pl


def kernel(x, edge_attr, l0_W1, l0_b1, l0_W2, l0_b2, l0_gamma, l0_beta, l1_W1, l1_b1, l1_W2, l1_b2, l1_gamma, l1_beta, edge_index, frag_batch, graph_batch):
    raise NotImplementedError("write your pallas kernel here")


# speedup vs baseline: 1.5980x; 1.5980x over previous
import functools

import jax
import jax.numpy as jnp
from jax import lax
from jax.experimental import pallas as pl
from jax.experimental.pallas import tpu as pltpu
from jax.experimental.pallas import tpu_sc as plsc

N = 10000
E = 160000
DIN = 16
DEMB = 64
NFRAG = 512
NGRAPH = 64
EPS = 1e-5

NC = 2
NS = 16
NW = NC * NS

CHUNK = 128
NCHUNKS = E // CHUNK
CPW = -(-NCHUNKS // NW)
PAD_ROWS = NW * CPW

ROWS_PER_SUB = N // NS

TE = 2000
NTILES = E // TE


def _sc_gather_body(table_hbm, idx_hbm, out_hbm, idx_v, rows_v, sem):
    wid = lax.axis_index("s") * NC + lax.axis_index("c")
    base = wid * CPW
    pltpu.sync_copy(idx_hbm.at[pl.ds(base, CPW)], idx_v)
    nt = jnp.minimum(CPW, NCHUNKS - base)

    def body(t, carry):
        c = base + t
        pltpu.async_copy(table_hbm.at[idx_v.at[t]], rows_v, sem).wait()
        pltpu.sync_copy(rows_v, out_hbm.at[pl.ds(c * CHUNK, CHUNK)])
        return carry

    lax.fori_loop(0, nt, body, 0)


def _sc_gather(table, idx2d):
    mesh = plsc.VectorSubcoreMesh(core_axis_name="c", subcore_axis_name="s")
    f = pl.kernel(
        _sc_gather_body,
        mesh=mesh,
        compiler_params=pltpu.CompilerParams(use_tc_tiling_on_sc=False),
        out_type=jax.ShapeDtypeStruct((E, DIN), jnp.float32),
        scratch_types=[
            pltpu.VMEM((CPW, CHUNK), jnp.int32),
            pltpu.VMEM((CHUNK, DIN), jnp.float32),
            pltpu.SemaphoreType.DMA,
        ],
    )
    return f(table, idx2d)


def _sc_scatter_body(msg_hbm, idx_hbm, zeros_hbm, out_hbm,
                     idx_v, msg_v, accum_sh, sem):
    cid = lax.axis_index("c")
    sid = lax.axis_index("s")
    wid = sid * NC + cid
    pltpu.sync_copy(zeros_hbm.at[pl.ds(sid * ROWS_PER_SUB, ROWS_PER_SUB)],
                    accum_sh.at[pl.ds(sid * ROWS_PER_SUB, ROWS_PER_SUB)])
    plsc.subcore_barrier()

    base = wid * CPW
    pltpu.sync_copy(idx_hbm.at[pl.ds(base, CPW)], idx_v)
    nt = jnp.minimum(CPW, NCHUNKS - base)

    def body(t, carry):
        c = base + t
        pltpu.sync_copy(msg_hbm.at[pl.ds(c * CHUNK, CHUNK)], msg_v)
        pltpu.sync_copy(msg_v, accum_sh.at[idx_v.at[t]], add=True)
        return carry

    lax.fori_loop(0, nt, body, 0)
    plsc.subcore_barrier()
    pltpu.sync_copy(accum_sh.at[pl.ds(sid * ROWS_PER_SUB, ROWS_PER_SUB)],
                    out_hbm.at[cid, pl.ds(sid * ROWS_PER_SUB, ROWS_PER_SUB)])


def _sc_scatter(msg, idx2d, zeros):
    mesh = plsc.VectorSubcoreMesh(core_axis_name="c", subcore_axis_name="s")
    f = pl.kernel(
        _sc_scatter_body,
        mesh=mesh,
        compiler_params=pltpu.CompilerParams(use_tc_tiling_on_sc=False),
        out_type=jax.ShapeDtypeStruct((NC, N, DIN), jnp.float32),
        scratch_types=[
            pltpu.VMEM((CPW, CHUNK), jnp.int32),
            pltpu.VMEM((CHUNK, DIN), jnp.float32),
            pltpu.VMEM_SHARED((N, DIN), jnp.float32),
            pltpu.SemaphoreType.DMA,
        ],
    )
    return f(msg, idx2d, zeros)


def _mlp_body(ea_ref, xj_ref, W1_ref, b1_ref, W2_ref, b2_ref, out_ref):
    ea = ea_ref[...]
    xj = xj_ref[...]
    h = jnp.maximum(
        jnp.dot(ea.astype(jnp.bfloat16), W1_ref[...].astype(jnp.bfloat16),
                preferred_element_type=jnp.float32) + b1_ref[...], 0.0)
    Wr = jnp.dot(h.astype(jnp.bfloat16), W2_ref[...].astype(jnp.bfloat16),
                 preferred_element_type=jnp.float32) + b2_ref[...]
    ri = lax.broadcasted_iota(jnp.int32, (DIN, DIN * DIN), 0)
    rj = lax.broadcasted_iota(jnp.int32, (DIN, DIN * DIN), 1)
    R = (ri == rj // DIN).astype(jnp.float32)
    xrep = jnp.dot(xj, R, preferred_element_type=jnp.float32, precision=lax.Precision.HIGHEST)
    sj = lax.broadcasted_iota(jnp.int32, (DIN * DIN, DIN), 0)
    so = lax.broadcasted_iota(jnp.int32, (DIN * DIN, DIN), 1)
    S = ((sj % DIN) == so).astype(jnp.float32)
    prod = (xrep.astype(jnp.bfloat16).astype(jnp.float32)
            * Wr.astype(jnp.bfloat16).astype(jnp.float32))
    out_ref[...] = jnp.dot(prod, S, preferred_element_type=jnp.float32, precision=lax.Precision.HIGHEST)


def _tc_edge_mlp(ea, xj, W1, b1, W2, b2):
    return pl.pallas_call(
        _mlp_body,
        grid=(NTILES,),
        in_specs=[
            pl.BlockSpec((TE, DIN), lambda i: (i, 0)),
            pl.BlockSpec((TE, DIN), lambda i: (i, 0)),
            pl.BlockSpec((DIN, DEMB), lambda i: (0, 0)),
            pl.BlockSpec((1, DEMB), lambda i: (0, 0)),
            pl.BlockSpec((DEMB, DIN * DIN), lambda i: (0, 0)),
            pl.BlockSpec((1, DIN * DIN), lambda i: (0, 0)),
        ],
        out_specs=pl.BlockSpec((TE, DIN), lambda i: (i, 0)),
        out_shape=jax.ShapeDtypeStruct((E, DIN), jnp.float32),
    )(ea, xj, W1, b1.reshape(1, DEMB), W2, b2.reshape(1, DIN * DIN))


def _bn_math(p0, p1, gamma, beta):
    h = jnp.maximum(p0 + p1, 0.0)
    mu = jnp.mean(h, axis=0, keepdims=True)
    var = jnp.mean((h - mu) ** 2, axis=0, keepdims=True)
    return (h - mu) * lax.rsqrt(var + EPS) * gamma + beta


def _bn_body(p_ref, g_ref, b_ref, out_ref):
    out_ref[...] = _bn_math(p_ref[0], p_ref[1], g_ref[...], b_ref[...])


def _tc_bn(p, gamma, beta):
    return pl.pallas_call(
        _bn_body,
        in_specs=[
            pl.BlockSpec((NC, N, DIN), lambda: (0, 0, 0)),
            pl.BlockSpec((1, DIN), lambda: (0, 0)),
            pl.BlockSpec((1, DIN), lambda: (0, 0)),
        ],
        out_specs=pl.BlockSpec((N, DIN), lambda: (0, 0)),
        out_shape=jax.ShapeDtypeStruct((N, DIN), jnp.float32),
    )(p, gamma.reshape(1, DIN), beta.reshape(1, DIN))


_AGG_CHUNK = 1000


def _segment_mean(h, batch_ref, nseg):
    s = jnp.zeros((nseg, DIN), jnp.float32)
    cnt = jnp.zeros((nseg, 1), jnp.float32)
    for c in range(N // _AGG_CHUNK):
        hc = h[c * _AGG_CHUNK:(c + 1) * _AGG_CHUNK]
        bc = batch_ref[:, c * _AGG_CHUNK:(c + 1) * _AGG_CHUNK]
        oh = (lax.broadcasted_iota(jnp.int32, (nseg, _AGG_CHUNK), 0)
              == bc).astype(jnp.float32)
        s = s + lax.dot_general(oh, hc, (((1,), (0,)), ((), ())),
                                preferred_element_type=jnp.float32, precision=lax.Precision.HIGHEST)
        cnt = cnt + jnp.sum(oh, axis=1, keepdims=True)
    return jnp.where(cnt > 0, s / jnp.where(cnt > 0, cnt, 1.0), 0.0)


def _agg_body(q_ref, g_ref, b_ref, frag_ref, graph_ref, outf_ref, outg_ref):
    h = _bn_math(q_ref[0], q_ref[1], g_ref[...], b_ref[...])
    outf_ref[...] = _segment_mean(h, frag_ref, NFRAG)
    outg_ref[...] = _segment_mean(h, graph_ref, NGRAPH)


def _tc_bn_agg(q, gamma, beta, frag2d, graph2d):
    return pl.pallas_call(
        _agg_body,
        in_specs=[
            pl.BlockSpec((NC, N, DIN), lambda: (0, 0, 0)),
            pl.BlockSpec((1, DIN), lambda: (0, 0)),
            pl.BlockSpec((1, DIN), lambda: (0, 0)),
            pl.BlockSpec((1, N), lambda: (0, 0)),
            pl.BlockSpec((1, N), lambda: (0, 0)),
        ],
        out_specs=[
            pl.BlockSpec((NFRAG, DIN), lambda: (0, 0)),
            pl.BlockSpec((NGRAPH, DIN), lambda: (0, 0)),
        ],
        out_shape=[
            jax.ShapeDtypeStruct((NFRAG, DIN), jnp.float32),
            jax.ShapeDtypeStruct((NGRAPH, DIN), jnp.float32),
        ],
    )(q, gamma.reshape(1, DIN), beta.reshape(1, DIN), frag2d, graph2d)


def kernel(x, edge_attr, l0_W1, l0_b1, l0_W2, l0_b2, l0_gamma, l0_beta,
           l1_W1, l1_b1, l1_W2, l1_b2, l1_gamma, l1_beta,
           edge_index, frag_batch, graph_batch):
    src2d = jnp.pad(edge_index[0].reshape(NCHUNKS, CHUNK),
                    ((0, PAD_ROWS - NCHUNKS), (0, 0)))
    dst2d = jnp.pad(edge_index[1].reshape(NCHUNKS, CHUNK),
                    ((0, PAD_ROWS - NCHUNKS), (0, 0)))
    zeros = jnp.zeros((N, DIN), jnp.float32)

    xj1 = _sc_gather(x, src2d)
    msg1 = _tc_edge_mlp(edge_attr, xj1, l0_W1, l0_b1, l0_W2, l0_b2)
    p1 = _sc_scatter(msg1, dst2d, zeros)
    h1 = _tc_bn(p1, l0_gamma, l0_beta)

    xj2 = _sc_gather(h1, src2d)
    msg2 = _tc_edge_mlp(edge_attr, xj2, l1_W1, l1_b1, l1_W2, l1_b2)
    p2 = _sc_scatter(msg2, dst2d, zeros)

    out_f, out_g = _tc_bn_agg(p2, l1_gamma, l1_beta,
                              frag_batch.reshape(1, N),
                              graph_batch.reshape(1, N))
    return (out_f, out_g)
